# baseline (speedup 1.0000x reference)
import functools

import jax
import jax.numpy as jnp
from jax.experimental import pallas as pl
from jax.experimental.pallas import tpu as pltpu


def _mlp_head_kernel(x_ref, w1_ref, w234_ref, b_ref, wh_ref, bh_ref,
                     bc_ref, fc_ref, *, Lb, chunk):
    nt = (((1,), (1,)), ((), ()))
    w1 = w1_ref[...].astype(jnp.bfloat16)
    w2 = w234_ref[0].astype(jnp.bfloat16)
    w3 = w234_ref[1].astype(jnp.bfloat16)
    w4 = w234_ref[2].astype(jnp.bfloat16)
    wh = wh_ref[...].astype(jnp.bfloat16)

    def body(c, carry):
        sl = pl.ds(c * chunk, chunk)
        h = x_ref[sl, :].astype(jnp.bfloat16)
        h = jax.lax.dot_general(h, w1, nt,
                                preferred_element_type=jnp.float32)
        h = jnp.maximum(h + b_ref[0], 0.0).astype(jnp.bfloat16)
        for i, w in enumerate((w2, w3, w4), start=1):
            h = jax.lax.dot_general(h, w, nt,
                                    preferred_element_type=jnp.float32)
            h = jnp.maximum(h + b_ref[i], 0.0).astype(jnp.bfloat16)
        out = jax.lax.dot_general(h, wh, nt,
                                  preferred_element_type=jnp.float32)
        out = out + bh_ref[...]
        bc_ref[sl, :] = out[:, :Lb]
        fc_ref[sl, :] = out[:, Lb:]
        return carry

    jax.lax.fori_loop(0, x_ref.shape[0] // chunk, body, 0)


def kernel(x, w1_t, w234_t, b1234, wh_t, bh):
    B, Lb = x.shape
    Lf = wh_t.shape[0] - Lb

    b = jnp.transpose(b1234, (0, 2, 1))
    bh_row = bh.T

    tile_b = 4096 if B % 4096 == 0 else B
    chunk = 2048 if tile_b % 2048 == 0 else tile_b

    in_specs = [
        pl.BlockSpec((tile_b, Lb), lambda i: (i, 0)),
        pl.BlockSpec(w1_t.shape, lambda i: (0, 0)),
        pl.BlockSpec(w234_t.shape, lambda i: (0, 0, 0)),
        pl.BlockSpec(b.shape, lambda i: (0, 0, 0)),
        pl.BlockSpec(wh_t.shape, lambda i: (0, 0)),
        pl.BlockSpec(bh_row.shape, lambda i: (0, 0)),
    ]
    out_specs = (
        pl.BlockSpec((tile_b, Lb), lambda i: (i, 0)),
        pl.BlockSpec((tile_b, Lf), lambda i: (i, 0)),
    )

    return pl.pallas_call(
        functools.partial(_mlp_head_kernel, Lb=Lb, chunk=chunk),
        out_shape=(jax.ShapeDtypeStruct((B, Lb), jnp.float32),
                   jax.ShapeDtypeStruct((B, Lf), jnp.float32)),
        grid=(B // tile_b,),
        in_specs=in_specs,
        out_specs=out_specs,
        compiler_params=pltpu.CompilerParams(
            dimension_semantics=("parallel",),
            vmem_limit_bytes=60 * 1024 * 1024,
            disable_bounds_checks=True,
            disable_semaphore_checks=True),
    )(x, w1_t, w234_t, b, wh_t, bh_row)


# speedup vs baseline: 1.0674x; 1.0674x over previous
import functools

import jax
import jax.numpy as jnp
from jax.experimental import pallas as pl
from jax.experimental.pallas import tpu as pltpu


def _mlp_head_kernel(x_ref, w1_ref, w234_ref, b_ref, wh_ref, bh_ref,
                     bc_ref, fc_ref, *, Lb):
    nt = (((1,), (1,)), ((), ()))
    xb = x_ref[...].astype(jnp.bfloat16)
    h = jax.lax.dot_general(xb, w1_ref[...].astype(jnp.bfloat16), nt,
                            preferred_element_type=jnp.float32)
    h = jnp.maximum(h + b_ref[0], 0.0).astype(jnp.bfloat16)
    for i in range(3):
        h = jax.lax.dot_general(h, w234_ref[i].astype(jnp.bfloat16), nt,
                                preferred_element_type=jnp.float32)
        h = jnp.maximum(h + b_ref[i + 1], 0.0).astype(jnp.bfloat16)
    out = jax.lax.dot_general(h, wh_ref[...].astype(jnp.bfloat16), nt,
                              preferred_element_type=jnp.float32)
    out = out + bh_ref[...]
    bc_ref[...] = out[:, :Lb]
    fc_ref[...] = out[:, Lb:]


def kernel(x, w1_t, w234_t, b1234, wh_t, bh):
    B, Lb = x.shape
    Lf = wh_t.shape[0] - Lb

    b = jnp.transpose(b1234, (0, 2, 1))
    bh_row = bh.T

    tile_b = 2048 if B % 2048 == 0 else B

    in_specs = [
        pl.BlockSpec((tile_b, Lb), lambda i: (i, 0)),
        pl.BlockSpec(w1_t.shape, lambda i: (0, 0)),
        pl.BlockSpec(w234_t.shape, lambda i: (0, 0, 0)),
        pl.BlockSpec(b.shape, lambda i: (0, 0, 0)),
        pl.BlockSpec(wh_t.shape, lambda i: (0, 0)),
        pl.BlockSpec(bh_row.shape, lambda i: (0, 0)),
    ]
    out_specs = (
        pl.BlockSpec((tile_b, Lb), lambda i: (i, 0)),
        pl.BlockSpec((tile_b, Lf), lambda i: (i, 0)),
    )

    return pl.pallas_call(
        functools.partial(_mlp_head_kernel, Lb=Lb),
        out_shape=(jax.ShapeDtypeStruct((B, Lb), jnp.float32),
                   jax.ShapeDtypeStruct((B, Lf), jnp.float32)),
        grid=(B // tile_b,),
        in_specs=in_specs,
        out_specs=out_specs,
        compiler_params=pltpu.CompilerParams(
            dimension_semantics=("parallel",),
            vmem_limit_bytes=60 * 1024 * 1024,
            disable_bounds_checks=True,
            disable_semaphore_checks=True),
    )(x, w1_t, w234_t, b, wh_t, bh_row)
